# baseline (speedup 1.0000x reference)
import jax
import jax.numpy as jnp
from jax.experimental import pallas as pl
from jax.experimental.pallas import tpu as pltpu

_B, _H, _W, _D = 8, 128, 128, 64
_NS = 2
_NB = _B // _NS
_HH = _H // 2


def _masks_of(s0):
    m1 = jnp.logical_and(s0 > 1000.0, s0 < 3000.0).astype(jnp.float32)
    m2 = jnp.logical_or(
        jnp.logical_and(s0 > 0.0, s0 < 1000.0), s0 > 3000.0
    ).astype(jnp.float32)
    return m1, m2


def _sums_body(*refs):
    yt_refs = refs[:_NS]
    yp_refs = refs[_NS:2 * _NS]
    acc1_ref, acc2_ref, cnt_ref = refs[2 * _NS:]
    b = pl.program_id(0)

    @pl.when(b == 0)
    def _zero():
        acc1_ref[...] = jnp.zeros((_H, _W), jnp.float32)
        acc2_ref[...] = jnp.zeros((_H, _W), jnp.float32)
        cnt_ref[...] = jnp.zeros((1, 128), jnp.float32)

    hh = b % 2
    p1 = jnp.zeros((_HH, _W), jnp.float32)
    p2 = jnp.zeros((_HH, _W), jnp.float32)
    c1 = jnp.float32(0.0)
    c2 = jnp.float32(0.0)
    for yt_ref, yp_ref in zip(yt_refs, yp_refs):
        m1, m2 = _masks_of(yt_ref[0, 0])
        ypv = yp_ref[0]
        p1 = p1 + jnp.sum(ypv * m1[None], axis=1)
        p2 = p2 + jnp.sum(ypv * m2[None], axis=1)
        c1 = c1 + jnp.sum(m1)
        c2 = c2 + jnp.sum(m2)

    sl = pl.ds(hh * _HH, _HH)
    acc1_ref[sl, :] += p1
    acc2_ref[sl, :] += p2

    @pl.when(hh == 0)
    def _cnt():
        lane = jax.lax.broadcasted_iota(jnp.int32, (1, 128), 1)
        cnt_ref[...] += (
            jnp.where(lane == 0, c1, 0.0) + jnp.where(lane == 1, c2, 0.0)
        )


def _corr_body(acc1_ref, acc2_ref, cnt_ref, out_ref):
    a = jnp.sum(acc1_ref[...], axis=1, keepdims=True) / cnt_ref[0, 0]
    bb = jnp.sum(acc2_ref[...], axis=1, keepdims=True) / cnt_ref[0, 1]
    am = a - jnp.mean(a)
    bm = bb - jnp.mean(bb)
    cov = jnp.mean(am * bm)
    sx = jnp.sqrt(jnp.mean(am * am))
    sy = jnp.sqrt(jnp.mean(bm * bm))
    corr = cov / (sx * sy)
    out_ref[...] = jnp.abs(jnp.clip(corr, -1.0, 1.0)).reshape(1, 1)


def kernel(y_true, y_pred):
    yt = jnp.transpose(y_true[..., 0], (0, 1, 3, 2))
    yp = jnp.transpose(y_pred[..., 0], (0, 1, 3, 2))
    acc1, acc2, cnt = pl.pallas_call(
        _sums_body,
        grid=(2 * _NB,),
        in_specs=(
            [pl.BlockSpec((1, 1, _D, _W),
                          (lambda s: lambda b: (b // 2 + s * _NB, 0, 0, 0))(s))
             for s in range(_NS)]
            + [pl.BlockSpec((1, _HH, _D, _W),
                            (lambda s: lambda b: (b // 2 + s * _NB, b % 2, 0, 0))(s))
               for s in range(_NS)]
        ),
        out_specs=[
            pl.BlockSpec((_H, _W), lambda b: (0, 0)),
            pl.BlockSpec((_H, _W), lambda b: (0, 0)),
            pl.BlockSpec((1, 128), lambda b: (0, 0)),
        ],
        out_shape=[
            jax.ShapeDtypeStruct((_H, _W), jnp.float32),
            jax.ShapeDtypeStruct((_H, _W), jnp.float32),
            jax.ShapeDtypeStruct((1, 128), jnp.float32),
        ],
    )(*([yt] * _NS + [yp] * _NS))
    out = pl.pallas_call(
        _corr_body,
        out_shape=jax.ShapeDtypeStruct((1, 1), jnp.float32),
    )(acc1, acc2, cnt)
    return out


# speedup vs baseline: 1.0313x; 1.0313x over previous
import jax
import jax.numpy as jnp
from jax.experimental import pallas as pl
from jax.experimental.pallas import tpu as pltpu

_B, _H, _W, _D = 8, 128, 128, 64
_NS = 2
_NB = _B // _NS


def _masks_of(s0):
    m1 = jnp.logical_and(s0 > 1000.0, s0 < 3000.0).astype(jnp.float32)
    m2 = jnp.logical_or(
        jnp.logical_and(s0 > 0.0, s0 < 1000.0), s0 > 3000.0
    ).astype(jnp.float32)
    return m1, m2


def _sums_body(*refs):
    yt_refs = refs[:_NS]
    yp_refs = refs[_NS:2 * _NS]
    acc1_ref, acc2_ref, cnt_ref = refs[2 * _NS:]
    b = pl.program_id(0)

    @pl.when(b == 0)
    def _zero():
        acc1_ref[...] = jnp.zeros((_H, _W), jnp.float32)
        acc2_ref[...] = jnp.zeros((_H, _W), jnp.float32)
        cnt_ref[...] = jnp.zeros((1, 128), jnp.float32)

    p1 = jnp.zeros((_H, _W), jnp.float32)
    p2 = jnp.zeros((_H, _W), jnp.float32)
    c1 = jnp.float32(0.0)
    c2 = jnp.float32(0.0)
    for yt_ref, yp_ref in zip(yt_refs, yp_refs):
        m1, m2 = _masks_of(yt_ref[0, 0])
        ypv = yp_ref[0]
        p1 = p1 + jnp.sum(ypv * m1[None], axis=1)
        p2 = p2 + jnp.sum(ypv * m2[None], axis=1)
        c1 = c1 + jnp.sum(m1)
        c2 = c2 + jnp.sum(m2)

    acc1_ref[...] += p1
    acc2_ref[...] += p2

    lane = jax.lax.broadcasted_iota(jnp.int32, (1, 128), 1)
    cnt_ref[...] += jnp.where(lane == 0, c1, 0.0) + jnp.where(lane == 1, c2, 0.0)


def _corr_body(acc1_ref, acc2_ref, cnt_ref, out_ref):
    a = jnp.sum(acc1_ref[...], axis=1, keepdims=True) / cnt_ref[0, 0]
    bb = jnp.sum(acc2_ref[...], axis=1, keepdims=True) / cnt_ref[0, 1]
    am = a - jnp.mean(a)
    bm = bb - jnp.mean(bb)
    cov = jnp.mean(am * bm)
    sx = jnp.sqrt(jnp.mean(am * am))
    sy = jnp.sqrt(jnp.mean(bm * bm))
    corr = cov / (sx * sy)
    out_ref[...] = jnp.abs(jnp.clip(corr, -1.0, 1.0)).reshape(1, 1)


def kernel(y_true, y_pred):
    yt = jnp.transpose(y_true[..., 0], (0, 1, 3, 2))
    yp = jnp.transpose(y_pred[..., 0], (0, 1, 3, 2))
    acc1, acc2, cnt = pl.pallas_call(
        _sums_body,
        grid=(_NB,),
        in_specs=(
            [pl.BlockSpec((1, 1, _D, _W),
                          (lambda s: lambda b: (b + s * _NB, 0, 0, 0))(s))
             for s in range(_NS)]
            + [pl.BlockSpec((1, _H, _D, _W),
                            (lambda s: lambda b: (b + s * _NB, 0, 0, 0))(s))
               for s in range(_NS)]
        ),
        out_specs=[
            pl.BlockSpec((_H, _W), lambda b: (0, 0)),
            pl.BlockSpec((_H, _W), lambda b: (0, 0)),
            pl.BlockSpec((1, 128), lambda b: (0, 0)),
        ],
        out_shape=[
            jax.ShapeDtypeStruct((_H, _W), jnp.float32),
            jax.ShapeDtypeStruct((_H, _W), jnp.float32),
            jax.ShapeDtypeStruct((1, 128), jnp.float32),
        ],
    )(*([yt] * _NS + [yp] * _NS))
    out = pl.pallas_call(
        _corr_body,
        out_shape=jax.ShapeDtypeStruct((1, 1), jnp.float32),
    )(acc1, acc2, cnt)
    return out
